# baseline (speedup 1.0000x reference)
import functools

import jax
import jax.numpy as jnp
from jax import lax
from jax.experimental import pallas as pl
from jax.experimental.pallas import tpu as pltpu
from jax.experimental.pallas import tpu_sc as plsc

_B = 128
_DIN = 2048
_DHID = 65536
_K = 32
_BH = 1024
_NBLK = _DHID // _BH
_CHUNK = 128
_NCHUNK = _DHID // _CHUNK
_MBH = 16384
_NW = 32
_RPW = _B // _NW


def _encode_body(x_ref, w_ref, be_ref, z_ref):
    z = lax.dot_general(x_ref[...], w_ref[...], (((1,), (1,)), ((), ())),
                        preferred_element_type=jnp.float32)
    z_ref[...] = jnp.maximum(z + be_ref[...], 0.0)


_encode = pl.pallas_call(
    _encode_body,
    grid=(_NBLK,),
    in_specs=[
        pl.BlockSpec((_B, _DIN), lambda i: (0, 0)),
        pl.BlockSpec((_BH, _DIN), lambda i: (i, 0)),
        pl.BlockSpec((1, _BH), lambda i: (0, i)),
    ],
    out_specs=pl.BlockSpec((_B, _BH), lambda i: (0, i)),
    out_shape=jax.ShapeDtypeStruct((_B, _DHID), jnp.float32),
)


def _chunkmax_body(z_ref, m_ref):
    zb = z_ref[...]
    m_ref[...] = jnp.max(zb.reshape(_B, _MBH // _CHUNK, _CHUNK), axis=2)


_chunkmax = pl.pallas_call(
    _chunkmax_body,
    grid=(_DHID // _MBH,),
    in_specs=[pl.BlockSpec((_B, _MBH), lambda i: (0, i))],
    out_specs=pl.BlockSpec((_B, _MBH // _CHUNK), lambda i: (0, i)),
    out_shape=jax.ShapeDtypeStruct((_B, _NCHUNK), jnp.float32),
)


def _select32(read, nv, hist_ref, s_ref, rank):
    zeros16 = jnp.zeros((16,), jnp.int32)
    pfx_vec = jnp.zeros((16,), jnp.int32)
    r_vec = jnp.full((16,), rank, jnp.int32)
    for level in range(4):
        sh = 24 - 8 * level

        for j in range(16):
            hist_ref[pl.ds(j * 16, 16)] = zeros16

        pfx = pfx_vec

        def _data(i, _):
            v = read(i)
            u = jnp.maximum(lax.bitcast_convert_type(v, jnp.int32), 0)
            bin_ = (u >> sh) & 0xFF
            if level == 0:
                cnt, last = plsc.scan_count(bin_)
            else:
                cnt, last = plsc.scan_count(bin_, (u >> (sh + 8)) == pfx)
            plsc.addupdate_scatter(hist_ref, [bin_], cnt, mask=last)
            return 0
        lax.fori_loop(0, nv, _data, 0, unroll=8)

        s_ref[pl.ds(256, 16)] = zeros16

        def _sfx(jj, carry):
            cnt_vec, csum_vec = carry
            j = 15 - jj
            tot = hist_ref[pl.ds(j * 16, 16)]
            sj = jnp.flip(jnp.cumsum(jnp.flip(tot))) + csum_vec
            s_ref[pl.ds(j * 16, 16)] = sj
            cnt = plsc.all_reduce_population_count(sj >= r_vec)
            new_csum = plsc.load_gather(s_ref,
                                        [jnp.full((16,), j * 16, jnp.int32)])
            return (cnt_vec + cnt, new_csum)
        cnt_vec, _ = lax.fori_loop(0, 16, _sfx, (zeros16, zeros16), unroll=4)
        b_vec = cnt_vec - 1
        sb1 = plsc.load_gather(s_ref, [b_vec + 1])
        r_vec = r_vec - sb1
        pfx_vec = (pfx_vec << 8) | b_vec
    return pfx_vec


_sc_mesh = plsc.VectorSubcoreMesh(core_axis_name="c", subcore_axis_name="s")


@functools.partial(
    pl.kernel,
    out_type=jax.ShapeDtypeStruct((_B, 16), jnp.int32),
    mesh=_sc_mesh,
    compiler_params=pltpu.CompilerParams(needs_layout_passes=False),
    scratch_types=[
        pltpu.VMEM((_NCHUNK,), jnp.float32),
        pltpu.VMEM((544,), jnp.int32),
        pltpu.VMEM((_K, _CHUNK), jnp.float32),
        pltpu.VMEM((256,), jnp.int32),
        pltpu.VMEM((272,), jnp.int32),
        pltpu.VMEM((16,), jnp.int32),
        pltpu.SemaphoreType.DMA,
    ],
)
def _sc_select(m_hbm, z2_hbm, thr_hbm, m_v, idx_v, cand_v, hist_v, s_v,
               tbuf_v, sem):
    wid = lax.axis_index("s") * 2 + lax.axis_index("c")
    lane = lax.iota(jnp.int32, 16)

    def _row(rr, _):
        row = wid * _RPW + rr
        pltpu.sync_copy(m_hbm.at[row], m_v)
        tb = _select32(lambda i: m_v[pl.ds(i * 16, 16)], _NCHUNK // 16,
                       hist_v, s_v, _K)

        def _coll_strict(j, off):
            v = m_v[pl.ds(j * 16, 16)]
            u = jnp.maximum(lax.bitcast_convert_type(v, jnp.int32), 0)
            msk = u > tb
            gid = row * _NCHUNK + j * 16 + lane
            plsc.store_compressed(idx_v.at[pl.ds(off, 16)], gid, mask=msk)
            return off + jnp.sum(msk.astype(jnp.int32))
        off1 = lax.fori_loop(0, _NCHUNK // 16, _coll_strict, jnp.int32(0), unroll=4)

        def _coll_ties(j, off):
            v = m_v[pl.ds(j * 16, 16)]
            u = jnp.maximum(lax.bitcast_convert_type(v, jnp.int32), 0)
            msk = u == tb
            gid = row * _NCHUNK + j * 16 + lane
            plsc.store_compressed(idx_v.at[pl.ds(off, 16)], gid, mask=msk)
            return off + jnp.sum(msk.astype(jnp.int32))
        lax.fori_loop(0, _NCHUNK // 16, _coll_ties, off1, unroll=4)

        pltpu.async_copy(z2_hbm.at[idx_v.at[pl.ds(0, _K)]], cand_v, sem).wait()

        def _readc(i):
            return cand_v[i >> 3, pl.ds((i & 7) * 16, 16)]
        tbuf_v[...] = _select32(_readc, _K * _CHUNK // 16, hist_v, s_v, _K)
        pltpu.sync_copy(tbuf_v, thr_hbm.at[row])
        return 0

    lax.fori_loop(0, _RPW, _row, 0)


def _decode_body(z_ref, t_ref, w_ref, bd_ref, o_ref):
    i = pl.program_id(0)
    z = z_ref[...]
    t = t_ref[...][:, 0:1]
    zs = jnp.where(z >= t, z, 0.0)
    part = lax.dot_general(zs, w_ref[...], (((1,), (1,)), ((), ())),
                           preferred_element_type=jnp.float32)

    @pl.when(i == 0)
    def _():
        o_ref[...] = part + bd_ref[...]

    @pl.when(i > 0)
    def _():
        o_ref[...] += part


_decode = pl.pallas_call(
    _decode_body,
    grid=(_NBLK,),
    in_specs=[
        pl.BlockSpec((_B, _BH), lambda i: (0, i)),
        pl.BlockSpec((_B, 16), lambda i: (0, 0)),
        pl.BlockSpec((_DIN, _BH), lambda i: (0, i)),
        pl.BlockSpec((1, _DIN), lambda i: (0, 0)),
    ],
    out_specs=pl.BlockSpec((_B, _DIN), lambda i: (0, 0)),
    out_shape=jax.ShapeDtypeStruct((_B, _DIN), jnp.float32),
)


def kernel(x, W_enc, b_enc, W_dec, b_dec):
    z = _encode(x, W_enc, b_enc.reshape(1, _DHID))
    m = _chunkmax(z)
    thr_bits = _sc_select(m, z.reshape(_B * _NCHUNK, _CHUNK))
    thr = lax.bitcast_convert_type(thr_bits, jnp.float32)
    return _decode(z, thr, W_dec, b_dec.reshape(1, _DIN))


# speedup vs baseline: 1.0481x; 1.0481x over previous
import functools

import jax
import jax.numpy as jnp
from jax import lax
from jax.experimental import pallas as pl
from jax.experimental.pallas import tpu as pltpu
from jax.experimental.pallas import tpu_sc as plsc

_B = 128
_DIN = 2048
_DHID = 65536
_K = 32
_BH = 1024
_NBLK = _DHID // _BH
_CHUNK = 128
_NCHUNK = _DHID // _CHUNK
_MBH = 16384
_NW = 32
_RPW = _B // _NW


def _encode_body(x_ref, w_ref, be_ref, z_ref, m_ref):
    z = lax.dot_general(x_ref[...], w_ref[...], (((1,), (1,)), ((), ())),
                        preferred_element_type=jnp.float32)
    z = jnp.maximum(z + be_ref[...], 0.0)
    z_ref[...] = z
    m8 = jnp.max(z.reshape(_B, _BH // _CHUNK, _CHUNK), axis=2)
    m_ref[...] = jnp.broadcast_to(
        m8[:, :, None], (_B, _BH // _CHUNK, 16)).reshape(_B, 128)


_encode = pl.pallas_call(
    _encode_body,
    grid=(_NBLK,),
    in_specs=[
        pl.BlockSpec((_B, _DIN), lambda i: (0, 0)),
        pl.BlockSpec((_BH, _DIN), lambda i: (i, 0)),
        pl.BlockSpec((1, _BH), lambda i: (0, i)),
    ],
    out_specs=[
        pl.BlockSpec((_B, _BH), lambda i: (0, i)),
        pl.BlockSpec((_B, 128), lambda i: (0, i)),
    ],
    out_shape=[
        jax.ShapeDtypeStruct((_B, _DHID), jnp.float32),
        jax.ShapeDtypeStruct((_B, _NBLK * 128), jnp.float32),
    ],
)


def _select32(read, nv, hist_ref, s_ref, rank):
    lane = lax.iota(jnp.int32, 16)
    ones = jnp.ones((16,), jnp.int32)
    zeros16 = jnp.zeros((16,), jnp.int32)
    pfx_vec = jnp.zeros((16,), jnp.int32)
    r_vec = jnp.full((16,), rank, jnp.int32)
    for level in range(4):
        sh = 24 - 8 * level

        def _zero(j, _):
            hist_ref[pl.ds(j * 16, 16)] = zeros16
            return 0
        lax.fori_loop(0, 256, _zero, 0, unroll=8)

        pfx = pfx_vec

        def _data(i, _):
            v = read(i)
            u = jnp.maximum(lax.bitcast_convert_type(v, jnp.int32), 0)
            bin_ = (u >> sh) & 0xFF
            idx = lane * 256 + bin_
            if level == 0:
                plsc.addupdate_scatter(hist_ref, [idx], ones)
            else:
                plsc.addupdate_scatter(hist_ref, [idx], ones,
                                       mask=(u >> (sh + 8)) == pfx)
            return 0
        lax.fori_loop(0, nv, _data, 0, unroll=8)

        s_ref[pl.ds(256, 16)] = zeros16

        def _sfx(jj, carry):
            cnt_vec, csum_vec = carry
            j = 15 - jj
            tot = hist_ref[pl.ds(j * 16, 16)]
            for l in range(1, 16):
                tot = tot + hist_ref[pl.ds(l * 256 + j * 16, 16)]
            sj = jnp.flip(jnp.cumsum(jnp.flip(tot))) + csum_vec
            s_ref[pl.ds(j * 16, 16)] = sj
            cnt = plsc.all_reduce_population_count(sj >= r_vec)
            new_csum = plsc.load_gather(s_ref,
                                        [jnp.full((16,), j * 16, jnp.int32)])
            return (cnt_vec + cnt, new_csum)
        cnt_vec, _ = lax.fori_loop(0, 16, _sfx, (zeros16, zeros16), unroll=4)
        b_vec = cnt_vec - 1
        sb1 = plsc.load_gather(s_ref, [b_vec + 1])
        r_vec = r_vec - sb1
        pfx_vec = (pfx_vec << 8) | b_vec
    return pfx_vec


_sc_mesh = plsc.VectorSubcoreMesh(core_axis_name="c", subcore_axis_name="s")


@functools.partial(
    pl.kernel,
    out_type=jax.ShapeDtypeStruct((_B, 16), jnp.int32),
    mesh=_sc_mesh,
    compiler_params=pltpu.CompilerParams(needs_layout_passes=False),
    scratch_types=[
        pltpu.VMEM((_NBLK * 128,), jnp.float32),
        pltpu.VMEM((544,), jnp.int32),
        pltpu.VMEM((_K, _CHUNK), jnp.float32),
        pltpu.VMEM((16 * 256,), jnp.int32),
        pltpu.VMEM((272,), jnp.int32),
        pltpu.VMEM((16,), jnp.int32),
        pltpu.SemaphoreType.DMA,
    ],
)
def _sc_select(m_hbm, z2_hbm, thr_hbm, m_v, idx_v, cand_v, hist_v, s_v,
               tbuf_v, sem):
    wid = lax.axis_index("s") * 2 + lax.axis_index("c")
    lane = lax.iota(jnp.int32, 16)

    cpb = _BH // _CHUNK

    def _mpos(c):
        return (c // cpb) * 128 + (c % cpb) * (128 // cpb)

    def _mread(i):
        return plsc.load_gather(m_v, [_mpos(i * 16 + lane)])

    def _row(rr, _):
        row = wid * _RPW + rr
        pltpu.sync_copy(m_hbm.at[row], m_v)
        tb = _select32(_mread, _NCHUNK // 16, hist_v, s_v, _K)

        def _coll_strict(j, off):
            v = _mread(j)
            u = jnp.maximum(lax.bitcast_convert_type(v, jnp.int32), 0)
            msk = u > tb
            gid = row * _NCHUNK + j * 16 + lane
            plsc.store_compressed(idx_v.at[pl.ds(off, 16)], gid, mask=msk)
            return off + jnp.sum(msk.astype(jnp.int32))
        off1 = lax.fori_loop(0, _NCHUNK // 16, _coll_strict, jnp.int32(0), unroll=4)

        def _coll_ties(j, off):
            v = _mread(j)
            u = jnp.maximum(lax.bitcast_convert_type(v, jnp.int32), 0)
            msk = u == tb
            gid = row * _NCHUNK + j * 16 + lane
            plsc.store_compressed(idx_v.at[pl.ds(off, 16)], gid, mask=msk)
            return off + jnp.sum(msk.astype(jnp.int32))
        lax.fori_loop(0, _NCHUNK // 16, _coll_ties, off1, unroll=4)

        pltpu.async_copy(z2_hbm.at[idx_v.at[pl.ds(0, _K)]], cand_v, sem).wait()

        def _readc(i):
            return cand_v[i >> 3, pl.ds((i & 7) * 16, 16)]
        tbuf_v[...] = _select32(_readc, _K * _CHUNK // 16, hist_v, s_v, _K)
        pltpu.sync_copy(tbuf_v, thr_hbm.at[row])
        return 0

    lax.fori_loop(0, _RPW, _row, 0)


def _decode_body(z_ref, t_ref, w_ref, bd_ref, o_ref):
    i = pl.program_id(0)
    z = z_ref[...]
    t = t_ref[...][:, 0:1]
    zs = jnp.where(z >= t, z, 0.0)
    part = lax.dot_general(zs, w_ref[...], (((1,), (1,)), ((), ())),
                           preferred_element_type=jnp.float32)

    @pl.when(i == 0)
    def _():
        o_ref[...] = part + bd_ref[...]

    @pl.when(i > 0)
    def _():
        o_ref[...] += part


_decode = pl.pallas_call(
    _decode_body,
    grid=(_NBLK,),
    in_specs=[
        pl.BlockSpec((_B, _BH), lambda i: (0, i)),
        pl.BlockSpec((_B, 16), lambda i: (0, 0)),
        pl.BlockSpec((_DIN, _BH), lambda i: (0, i)),
        pl.BlockSpec((1, _DIN), lambda i: (0, 0)),
    ],
    out_specs=pl.BlockSpec((_B, _DIN), lambda i: (0, 0)),
    out_shape=jax.ShapeDtypeStruct((_B, _DIN), jnp.float32),
)


def kernel(x, W_enc, b_enc, W_dec, b_dec):
    z, m = _encode(x, W_enc, b_enc.reshape(1, _DHID))
    thr_bits = _sc_select(m, z.reshape(_B * _NCHUNK, _CHUNK))
    thr = lax.bitcast_convert_type(thr_bits, jnp.float32)
    return _decode(z, thr, W_dec, b_dec.reshape(1, _DIN))


# speedup vs baseline: 1.0657x; 1.0167x over previous
import functools

import jax
import jax.numpy as jnp
from jax import lax
from jax.experimental import pallas as pl
from jax.experimental.pallas import tpu as pltpu
from jax.experimental.pallas import tpu_sc as plsc

_B = 128
_DIN = 2048
_DHID = 65536
_K = 32
_BH = 1024
_NBLK = _DHID // _BH
_CHUNK = 128
_NCHUNK = _DHID // _CHUNK
_MBH = 16384
_NW = 32
_RPW = _B // _NW


def _encode_body(x_ref, w_ref, be_ref, z_ref, m_ref):
    z = lax.dot_general(x_ref[...], w_ref[...], (((1,), (1,)), ((), ())),
                        preferred_element_type=jnp.float32)
    z = jnp.maximum(z + be_ref[...], 0.0)
    z_ref[...] = z
    m8 = jnp.max(z.reshape(_B, _BH // _CHUNK, _CHUNK), axis=2)
    m_ref[...] = jnp.broadcast_to(
        m8[:, :, None], (_B, _BH // _CHUNK, 16)).reshape(_B, 128)


_encode = pl.pallas_call(
    _encode_body,
    grid=(_NBLK,),
    in_specs=[
        pl.BlockSpec((_B, _DIN), lambda i: (0, 0)),
        pl.BlockSpec((_BH, _DIN), lambda i: (i, 0)),
        pl.BlockSpec((1, _BH), lambda i: (0, i)),
    ],
    out_specs=[
        pl.BlockSpec((_B, _BH), lambda i: (0, i)),
        pl.BlockSpec((_B, 128), lambda i: (0, i)),
    ],
    out_shape=[
        jax.ShapeDtypeStruct((_B, _DHID), jnp.float32),
        jax.ShapeDtypeStruct((_B, _NBLK * 128), jnp.float32),
    ],
)


def _select32(read, nv, hist_ref, s_ref, rank, cbuf_ref=None):
    lane = lax.iota(jnp.int32, 16)
    ones = jnp.ones((16,), jnp.int32)
    zeros16 = jnp.zeros((16,), jnp.int32)
    pfx_vec = jnp.zeros((16,), jnp.int32)
    r_vec = jnp.full((16,), rank, jnp.int32)
    def _read_u(i):
        return jnp.maximum(lax.bitcast_convert_type(read(i), jnp.int32), 0)

    cur_read, cur_nv, cur_unroll = _read_u, nv, 8
    for level in range(4):
        sh = 24 - 8 * level

        def _zero(j, _):
            hist_ref[pl.ds(j * 16, 16)] = zeros16
            return 0
        lax.fori_loop(0, 256, _zero, 0, unroll=8)

        pfx = pfx_vec
        rd = cur_read

        def _data(i, _):
            u = rd(i)
            bin_ = (u >> sh) & 0xFF
            idx = lane * 256 + bin_
            if level == 0:
                plsc.addupdate_scatter(hist_ref, [idx], ones)
            else:
                plsc.addupdate_scatter(hist_ref, [idx], ones,
                                       mask=(u >> (sh + 8)) == pfx)
            return 0
        lax.fori_loop(0, cur_nv, _data, 0, unroll=cur_unroll)

        s_ref[pl.ds(256, 16)] = zeros16

        def _sfx(jj, carry):
            cnt_vec, csum_vec = carry
            j = 15 - jj
            tot = hist_ref[pl.ds(j * 16, 16)]
            for l in range(1, 16):
                tot = tot + hist_ref[pl.ds(l * 256 + j * 16, 16)]
            sj = jnp.flip(jnp.cumsum(jnp.flip(tot))) + csum_vec
            s_ref[pl.ds(j * 16, 16)] = sj
            cnt = plsc.all_reduce_population_count(sj >= r_vec)
            new_csum = plsc.load_gather(s_ref,
                                        [jnp.full((16,), j * 16, jnp.int32)])
            return (cnt_vec + cnt, new_csum)
        cnt_vec, _ = lax.fori_loop(0, 16, _sfx, (zeros16, zeros16), unroll=4)
        b_vec = cnt_vec - 1
        sb1 = plsc.load_gather(s_ref, [b_vec + 1])
        r_vec = r_vec - sb1
        pfx_vec = (pfx_vec << 8) | b_vec

        if level == 0 and cbuf_ref is not None:
            pfx0 = pfx_vec

            def _compact(i, off):
                u = _read_u(i)
                msk = (u >> 24) == pfx0
                plsc.store_compressed(cbuf_ref.at[pl.ds(off, 16)], u,
                                      mask=msk)
                return off + jnp.sum(msk.astype(jnp.int32))
            cnt0 = lax.fori_loop(0, nv, _compact, jnp.int32(0), unroll=8)
            cbuf_ref[pl.ds(cnt0, 16)] = jnp.full((16,), -1, jnp.int32)

            cur_read = lambda i: cbuf_ref[pl.ds(i * 16, 16)]
            cur_nv = (cnt0 + 15) >> 4
            cur_unroll = 1
    return pfx_vec


_sc_mesh = plsc.VectorSubcoreMesh(core_axis_name="c", subcore_axis_name="s")


@functools.partial(
    pl.kernel,
    out_type=jax.ShapeDtypeStruct((_B, 16), jnp.int32),
    mesh=_sc_mesh,
    compiler_params=pltpu.CompilerParams(needs_layout_passes=False),
    scratch_types=[
        pltpu.VMEM((_NBLK * 128,), jnp.float32),
        pltpu.VMEM((544,), jnp.int32),
        pltpu.VMEM((_K, _CHUNK), jnp.float32),
        pltpu.VMEM((16 * 256,), jnp.int32),
        pltpu.VMEM((272,), jnp.int32),
        pltpu.VMEM((16,), jnp.int32),
        pltpu.VMEM((_K * _CHUNK + 16,), jnp.int32),
        pltpu.SemaphoreType.DMA,
    ],
)
def _sc_select(m_hbm, z2_hbm, thr_hbm, m_v, idx_v, cand_v, hist_v, s_v,
               tbuf_v, cbuf_v, sem):
    wid = lax.axis_index("s") * 2 + lax.axis_index("c")
    lane = lax.iota(jnp.int32, 16)

    cpb = _BH // _CHUNK

    def _mpos(c):
        return (c // cpb) * 128 + (c % cpb) * (128 // cpb)

    def _mread(i):
        return plsc.load_gather(m_v, [_mpos(i * 16 + lane)])

    def _row(rr, _):
        row = wid * _RPW + rr
        pltpu.sync_copy(m_hbm.at[row], m_v)
        tb = _select32(_mread, _NCHUNK // 16, hist_v, s_v, _K)

        def _coll_strict(j, off):
            v = _mread(j)
            u = jnp.maximum(lax.bitcast_convert_type(v, jnp.int32), 0)
            msk = u > tb
            gid = row * _NCHUNK + j * 16 + lane
            plsc.store_compressed(idx_v.at[pl.ds(off, 16)], gid, mask=msk)
            return off + jnp.sum(msk.astype(jnp.int32))
        off1 = lax.fori_loop(0, _NCHUNK // 16, _coll_strict, jnp.int32(0), unroll=4)

        def _coll_ties(j, off):
            v = _mread(j)
            u = jnp.maximum(lax.bitcast_convert_type(v, jnp.int32), 0)
            msk = u == tb
            gid = row * _NCHUNK + j * 16 + lane
            plsc.store_compressed(idx_v.at[pl.ds(off, 16)], gid, mask=msk)
            return off + jnp.sum(msk.astype(jnp.int32))
        lax.fori_loop(0, _NCHUNK // 16, _coll_ties, off1, unroll=4)

        pltpu.async_copy(z2_hbm.at[idx_v.at[pl.ds(0, _K)]], cand_v, sem).wait()

        def _readc(i):
            return cand_v[i >> 3, pl.ds((i & 7) * 16, 16)]
        tbuf_v[...] = _select32(_readc, _K * _CHUNK // 16, hist_v, s_v, _K,
                                cbuf_ref=cbuf_v)
        pltpu.sync_copy(tbuf_v, thr_hbm.at[row])
        return 0

    lax.fori_loop(0, _RPW, _row, 0)


def _decode_body(z_ref, t_ref, w_ref, bd_ref, o_ref):
    i = pl.program_id(0)
    z = z_ref[...]
    t = t_ref[...][:, 0:1]
    zs = jnp.where(z >= t, z, 0.0)
    part = lax.dot_general(zs, w_ref[...], (((1,), (1,)), ((), ())),
                           preferred_element_type=jnp.float32)

    @pl.when(i == 0)
    def _():
        o_ref[...] = part + bd_ref[...]

    @pl.when(i > 0)
    def _():
        o_ref[...] += part


_decode = pl.pallas_call(
    _decode_body,
    grid=(_NBLK,),
    in_specs=[
        pl.BlockSpec((_B, _BH), lambda i: (0, i)),
        pl.BlockSpec((_B, 16), lambda i: (0, 0)),
        pl.BlockSpec((_DIN, _BH), lambda i: (0, i)),
        pl.BlockSpec((1, _DIN), lambda i: (0, 0)),
    ],
    out_specs=pl.BlockSpec((_B, _DIN), lambda i: (0, 0)),
    out_shape=jax.ShapeDtypeStruct((_B, _DIN), jnp.float32),
)


def kernel(x, W_enc, b_enc, W_dec, b_dec):
    z, m = _encode(x, W_enc, b_enc.reshape(1, _DHID))
    thr_bits = _sc_select(m, z.reshape(_B * _NCHUNK, _CHUNK))
    thr = lax.bitcast_convert_type(thr_bits, jnp.float32)
    return _decode(z, thr, W_dec, b_dec.reshape(1, _DIN))


# speedup vs baseline: 1.1356x; 1.0656x over previous
import functools

import jax
import jax.numpy as jnp
from jax import lax
from jax.experimental import pallas as pl
from jax.experimental.pallas import tpu as pltpu
from jax.experimental.pallas import tpu_sc as plsc

_B = 128
_DIN = 2048
_DHID = 65536
_K = 32
_BH = 1024
_NBLK = _DHID // _BH
_CHUNK = 128
_NCHUNK = _DHID // _CHUNK
_MBH = 16384
_NW = 32
_RPW = _B // _NW


def _encode_body(x_ref, w_ref, be_ref, z_ref, m_ref):
    z = lax.dot_general(x_ref[...], w_ref[...], (((1,), (1,)), ((), ())),
                        preferred_element_type=jnp.float32)
    z = jnp.maximum(z + be_ref[...], 0.0)
    z_ref[...] = z
    m8 = jnp.max(z.reshape(_B, _BH // _CHUNK, _CHUNK), axis=2)
    m_ref[...] = jnp.broadcast_to(
        m8[:, :, None], (_B, _BH // _CHUNK, 16)).reshape(_B, 128)


_encode = pl.pallas_call(
    _encode_body,
    grid=(_NBLK,),
    in_specs=[
        pl.BlockSpec((_B, _DIN), lambda i: (0, 0)),
        pl.BlockSpec((_BH, _DIN), lambda i: (i, 0)),
        pl.BlockSpec((1, _BH), lambda i: (0, i)),
    ],
    out_specs=[
        pl.BlockSpec((_B, _BH), lambda i: (0, i)),
        pl.BlockSpec((_B, 128), lambda i: (0, i)),
    ],
    out_shape=[
        jax.ShapeDtypeStruct((_B, _DHID), jnp.float32),
        jax.ShapeDtypeStruct((_B, _NBLK * 128), jnp.float32),
    ],
)


def _select32(read, nv, hist_ref, s_ref, rank, cbuf_ref=None):
    lane = lax.iota(jnp.int32, 16)
    ones = jnp.ones((16,), jnp.int32)
    zeros16 = jnp.zeros((16,), jnp.int32)
    pfx_vec = jnp.zeros((16,), jnp.int32)
    r_vec = jnp.full((16,), rank, jnp.int32)
    def _read_u(i):
        return jnp.maximum(lax.bitcast_convert_type(read(i), jnp.int32), 0)

    cur_read, cur_nv, cur_unroll = _read_u, nv, 8
    for level in range(4):
        sh = 24 - 8 * level

        def _zero(j, _):
            hist_ref[pl.ds(j * 16, 16)] = zeros16
            return 0
        lax.fori_loop(0, 256, _zero, 0, unroll=8)

        pfx = pfx_vec
        rd = cur_read

        def _data(i, _):
            u = rd(i)
            bin_ = (u >> sh) & 0xFF
            idx = lane * 256 + bin_
            if level == 0:
                plsc.addupdate_scatter(hist_ref, [idx], ones)
            else:
                plsc.addupdate_scatter(hist_ref, [idx], ones,
                                       mask=(u >> (sh + 8)) == pfx)
            return 0
        lax.fori_loop(0, cur_nv, _data, 0, unroll=cur_unroll)

        s_ref[pl.ds(256, 16)] = zeros16

        def _sfx(jj, carry):
            cnt_vec, csum_vec = carry
            j = 15 - jj
            tot = hist_ref[pl.ds(j * 16, 16)]
            for l in range(1, 16):
                tot = tot + hist_ref[pl.ds(l * 256 + j * 16, 16)]
            sj = jnp.flip(jnp.cumsum(jnp.flip(tot))) + csum_vec
            s_ref[pl.ds(j * 16, 16)] = sj
            cnt = plsc.all_reduce_population_count(sj >= r_vec)
            new_csum = plsc.load_gather(s_ref,
                                        [jnp.full((16,), j * 16, jnp.int32)])
            return (cnt_vec + cnt, new_csum)
        cnt_vec, _ = lax.fori_loop(0, 16, _sfx, (zeros16, zeros16), unroll=4)
        b_vec = cnt_vec - 1
        sb1 = plsc.load_gather(s_ref, [b_vec + 1])
        r_vec = r_vec - sb1
        pfx_vec = (pfx_vec << 8) | b_vec

        if level == 0 and cbuf_ref is not None:
            pfx0 = pfx_vec

            def _compact(i, off):
                u = _read_u(i)
                msk = (u >> 24) == pfx0
                plsc.store_compressed(cbuf_ref.at[pl.ds(off, 16)], u,
                                      mask=msk)
                return off + jnp.sum(msk.astype(jnp.int32))
            cnt0 = lax.fori_loop(0, nv, _compact, jnp.int32(0), unroll=8)
            cbuf_ref[pl.ds(cnt0, 16)] = jnp.full((16,), -1, jnp.int32)

            cur_read = lambda i: cbuf_ref[pl.ds(i * 16, 16)]
            cur_nv = (cnt0 + 15) >> 4
            cur_unroll = 1
    return pfx_vec


_sc_mesh = plsc.VectorSubcoreMesh(core_axis_name="c", subcore_axis_name="s")


@functools.partial(
    pl.kernel,
    out_type=jax.ShapeDtypeStruct((_B, 16), jnp.int32),
    mesh=_sc_mesh,
    compiler_params=pltpu.CompilerParams(needs_layout_passes=False),
    scratch_types=[
        pltpu.VMEM((_NBLK * 128,), jnp.float32),
        pltpu.VMEM((544,), jnp.int32),
        pltpu.VMEM((_K, _CHUNK), jnp.float32),
        pltpu.VMEM((16 * 256,), jnp.int32),
        pltpu.VMEM((272,), jnp.int32),
        pltpu.VMEM((16,), jnp.int32),
        pltpu.VMEM((_K * _CHUNK + 16,), jnp.int32),
        pltpu.SemaphoreType.DMA,
    ],
)
def _sc_select(m_hbm, z_hbm, thr_hbm, m_v, idx_v, cand_v, hist_v, s_v,
               tbuf_v, cbuf_v, sem):
    wid = lax.axis_index("s") * 2 + lax.axis_index("c")
    lane = lax.iota(jnp.int32, 16)

    cpb = _BH // _CHUNK

    def _mpos(c):
        return (c // cpb) * 128 + (c % cpb) * (128 // cpb)

    def _mread(i):
        return plsc.load_gather(m_v, [_mpos(i * 16 + lane)])

    def _row(rr, _):
        row = wid * _RPW + rr
        pltpu.sync_copy(m_hbm.at[row], m_v)
        tb = _select32(_mread, _NCHUNK // 16, hist_v, s_v, _K)

        def _coll_strict(j, off):
            v = _mread(j)
            u = jnp.maximum(lax.bitcast_convert_type(v, jnp.int32), 0)
            msk = u > tb
            cid = j * 16 + lane
            plsc.store_compressed(idx_v.at[pl.ds(off, 16)], cid, mask=msk)
            return off + jnp.sum(msk.astype(jnp.int32))
        off1 = lax.fori_loop(0, _NCHUNK // 16, _coll_strict, jnp.int32(0), unroll=4)

        def _coll_ties(j, off):
            v = _mread(j)
            u = jnp.maximum(lax.bitcast_convert_type(v, jnp.int32), 0)
            msk = u == tb
            cid = j * 16 + lane
            plsc.store_compressed(idx_v.at[pl.ds(off, 16)], cid, mask=msk)
            return off + jnp.sum(msk.astype(jnp.int32))
        lax.fori_loop(0, _NCHUNK // 16, _coll_ties, off1, unroll=4)

        iv0 = idx_v[pl.ds(0, 16)]
        iv1 = idx_v[pl.ds(16, 16)]
        copies = []
        for k in range(_K):
            cid = (iv0 if k < 16 else iv1)[k % 16]
            copies.append(pltpu.make_async_copy(
                z_hbm.at[row, pl.ds(cid * _CHUNK, _CHUNK)], cand_v.at[k], sem))
        for c in copies:
            c.start()
        for c in copies:
            c.wait()

        def _readc(i):
            return cand_v[i >> 3, pl.ds((i & 7) * 16, 16)]
        tbuf_v[...] = _select32(_readc, _K * _CHUNK // 16, hist_v, s_v, _K,
                                cbuf_ref=cbuf_v)
        pltpu.sync_copy(tbuf_v, thr_hbm.at[row])
        return 0

    lax.fori_loop(0, _RPW, _row, 0)


def _decode_body(z_ref, t_ref, w_ref, bd_ref, o_ref):
    i = pl.program_id(0)
    z = z_ref[...]
    t = t_ref[...][:, 0:1]
    zs = jnp.where(z >= t, z, 0.0)
    part = lax.dot_general(zs, w_ref[...], (((1,), (1,)), ((), ())),
                           preferred_element_type=jnp.float32)

    @pl.when(i == 0)
    def _():
        o_ref[...] = part + bd_ref[...]

    @pl.when(i > 0)
    def _():
        o_ref[...] += part


_decode = pl.pallas_call(
    _decode_body,
    grid=(_NBLK,),
    in_specs=[
        pl.BlockSpec((_B, _BH), lambda i: (0, i)),
        pl.BlockSpec((_B, 16), lambda i: (0, 0)),
        pl.BlockSpec((_DIN, _BH), lambda i: (0, i)),
        pl.BlockSpec((1, _DIN), lambda i: (0, 0)),
    ],
    out_specs=pl.BlockSpec((_B, _DIN), lambda i: (0, 0)),
    out_shape=jax.ShapeDtypeStruct((_B, _DIN), jnp.float32),
)


def kernel(x, W_enc, b_enc, W_dec, b_dec):
    z, m = _encode(x, W_enc, b_enc.reshape(1, _DHID))
    thr_bits = _sc_select(m, z)
    thr = lax.bitcast_convert_type(thr_bits, jnp.float32)
    return _decode(z, thr, W_dec, b_dec.reshape(1, _DIN))
